# Initial kernel scaffold; baseline (speedup 1.0000x reference)
import jax
import jax.numpy as jnp
from jax.experimental import pallas as pl


def kernel(x, adj, W0, W1, W2, b):
    raise NotImplementedError("write your pallas kernel here")


# speedup vs baseline: 1.0466x; 1.0466x over previous
import jax
import jax.numpy as jnp
from jax.experimental import pallas as pl
from jax.experimental.pallas import tpu as pltpu


def _pick_block(n):
    for bm in (512, 400, 256, 200, 128, 80, 40, 16, 8):
        if n % bm == 0:
            return bm
    return 1


def _cheb_body(x_ref, adj_ref, w0_ref, w1_ref, w2_ref, b_ref,
               out_ref, t1_ref, p_ref):
    phase = pl.program_id(0)
    i = pl.program_id(1)
    bm = adj_ref.shape[0]
    rows = pl.ds(i * bm, bm)
    adj_blk = adj_ref[...]

    @pl.when(phase == 0)
    def _pass1():
        t1 = jnp.dot(adj_blk, x_ref[...], preferred_element_type=jnp.float32)
        t1_ref[rows, :] = t1
        x_blk = x_ref[rows, :]
        p_ref[rows, :] = (
            jnp.dot(x_blk, w0_ref[...] - w2_ref[...],
                    preferred_element_type=jnp.float32)
            + jnp.dot(t1, w1_ref[...], preferred_element_type=jnp.float32)
            + b_ref[...]
        )

    @pl.when(phase == 1)
    def _pass2():
        t2 = jnp.dot(adj_blk, t1_ref[...], preferred_element_type=jnp.float32)
        out_ref[...] = p_ref[rows, :] + jnp.dot(
            t2, 2.0 * w2_ref[...], preferred_element_type=jnp.float32)


def kernel(x, adj, W0, W1, W2, b):
    n, d_in = x.shape
    d_out = W0.shape[1]
    bm = _pick_block(n)
    nb = n // bm
    b2d = b.reshape(1, d_out).astype(jnp.float32)

    grid = (2, nb)
    out = pl.pallas_call(
        _cheb_body,
        grid=grid,
        in_specs=[
            pl.BlockSpec((n, d_in), lambda p, i: (0, 0)),
            pl.BlockSpec((bm, n), lambda p, i: (i, 0)),
            pl.BlockSpec((d_in, d_out), lambda p, i: (0, 0)),
            pl.BlockSpec((d_in, d_out), lambda p, i: (0, 0)),
            pl.BlockSpec((d_in, d_out), lambda p, i: (0, 0)),
            pl.BlockSpec((1, d_out), lambda p, i: (0, 0)),
        ],
        out_specs=pl.BlockSpec((bm, d_out), lambda p, i: (i, 0)),
        out_shape=jax.ShapeDtypeStruct((n, d_out), jnp.float32),
        scratch_shapes=[
            pltpu.VMEM((n, d_in), jnp.float32),
            pltpu.VMEM((n, d_out), jnp.float32),
        ],
        compiler_params=pltpu.CompilerParams(
            dimension_semantics=("arbitrary", "arbitrary"),
            vmem_limit_bytes=100 * 1024 * 1024,
        ),
    )(x, adj, W0, W1, W2, b2d)
    return out
